# baseline (speedup 1.0000x reference)
import jax
import jax.numpy as jnp
from jax.experimental import pallas as pl
from jax.experimental.pallas import tpu as pltpu

_BLOCK_J = 256


def _spmm_kernel(x_ref, w_ref, o_ref, xt_ref):
    @pl.when(pl.program_id(0) == 0)
    def _():
        xt_ref[...] = x_ref[...].T

    acc = jax.lax.dot_general(
        xt_ref[...], w_ref[...],
        dimension_numbers=(((1,), (0,)), ((), ())),
        preferred_element_type=jnp.float32,
    )
    o_ref[...] = acc.T


def kernel(input, W):
    size_in, cols = input.shape
    size_out = W.shape[1]
    grid = (size_out // _BLOCK_J,)
    return pl.pallas_call(
        _spmm_kernel,
        grid=grid,
        in_specs=[
            pl.BlockSpec((size_in, cols), lambda j: (0, 0)),
            pl.BlockSpec((size_in, _BLOCK_J), lambda j: (0, j)),
        ],
        out_specs=pl.BlockSpec((_BLOCK_J, cols), lambda j: (j, 0)),
        out_shape=jax.ShapeDtypeStruct((size_out, cols), jnp.float32),
        scratch_shapes=[pltpu.VMEM((cols, size_in), jnp.float32)],
        compiler_params=pltpu.CompilerParams(
            dimension_semantics=("arbitrary",),
        ),
    )(input, W)


# speedup vs baseline: 1.0717x; 1.0717x over previous
import jax
import jax.numpy as jnp
from jax.experimental import pallas as pl
from jax.experimental.pallas import tpu as pltpu

_BLOCK_J = 1024


def _spmm_kernel(x_ref, w_ref, o_ref, xt_ref):
    @pl.when(pl.program_id(0) == 0)
    def _():
        xt_ref[...] = x_ref[...].T

    acc = jax.lax.dot_general(
        xt_ref[...], w_ref[...],
        dimension_numbers=(((1,), (0,)), ((), ())),
        preferred_element_type=jnp.float32,
    )
    o_ref[...] = acc.T


def kernel(input, W):
    size_in, cols = input.shape
    size_out = W.shape[1]
    grid = (size_out // _BLOCK_J,)
    return pl.pallas_call(
        _spmm_kernel,
        grid=grid,
        in_specs=[
            pl.BlockSpec((size_in, cols), lambda j: (0, 0)),
            pl.BlockSpec((size_in, _BLOCK_J), lambda j: (0, j)),
        ],
        out_specs=pl.BlockSpec((_BLOCK_J, cols), lambda j: (j, 0)),
        out_shape=jax.ShapeDtypeStruct((size_out, cols), jnp.float32),
        scratch_shapes=[pltpu.VMEM((cols, size_in), jnp.float32)],
        compiler_params=pltpu.CompilerParams(
            dimension_semantics=("arbitrary",),
        ),
    )(input, W)
